# baseline (speedup 1.0000x reference)
import functools

import jax
import jax.numpy as jnp
from jax import lax
from jax.experimental import pallas as pl
from jax.experimental.pallas import tpu as pltpu
from jax.experimental.pallas import tpu_sc as plsc


def _sc_gather(emb_table, inpt, ffw_weight, trgs):
    B = inpt.shape[0]
    D = emb_table.shape[1]
    info = plsc.get_sparse_core_info()
    nc, ns = info.num_cores, info.num_subcores
    nw = nc * ns
    b_per_w = B // nw
    mesh = plsc.VectorSubcoreMesh(core_axis_name="c", subcore_axis_name="s")

    @functools.partial(
        pl.kernel,
        mesh=mesh,
        out_type=[
            jax.ShapeDtypeStruct((B, D), jnp.float32),
            jax.ShapeDtypeStruct((B, D), jnp.float32),
        ],
        scratch_types=[
            pltpu.VMEM((b_per_w,), jnp.int32),
            pltpu.VMEM((b_per_w, D), jnp.float32),
            pltpu.VMEM((b_per_w,), jnp.int32),
            pltpu.VMEM((b_per_w, D), jnp.float32),
            pltpu.SemaphoreType.DMA,
        ],
        compiler_params=pltpu.CompilerParams(use_tc_tiling_on_sc=False),
    )
    def gather_kernel(emb_hbm, inpt_hbm, ffw_hbm, trgs_hbm, e_out, c_out,
                      idx_e, rows_e, idx_c, rows_c, sem):
        wid = lax.axis_index("s") * nc + lax.axis_index("c")
        base = wid * b_per_w
        pltpu.sync_copy(inpt_hbm.at[pl.ds(base, b_per_w)], idx_e)
        pltpu.sync_copy(trgs_hbm.at[pl.ds(base, b_per_w)], idx_c)
        cp_e = pltpu.async_copy(emb_hbm.at[idx_e], rows_e, sem)
        cp_c = pltpu.async_copy(ffw_hbm.at[idx_c], rows_c, sem)
        cp_e.wait()
        cp_c.wait()
        pltpu.sync_copy(rows_e, e_out.at[pl.ds(base, b_per_w)])
        pltpu.sync_copy(rows_c, c_out.at[pl.ds(base, b_per_w)])

    return gather_kernel(emb_table, inpt, ffw_weight, trgs)


def _tc_loss(e, c, interpret=False):
    B, D = e.shape
    blk = 512
    scale = 1.0 / (B * B)

    def body(c_ref, e_ref, out_ref):
        i = pl.program_id(0)
        lgt = lax.dot_general(
            c_ref[...], e_ref[...],
            (((1,), (1,)), ((), ())),
            preferred_element_type=jnp.float32,
        )
        part = jnp.sum(jnp.log(1.0 + jnp.exp(-lgt))) * scale

        @pl.when(i == 0)
        def _():
            out_ref[0, 0] = 0.0

        out_ref[0, 0] += part

    out = pl.pallas_call(
        body,
        grid=(B // blk,),
        in_specs=[
            pl.BlockSpec((blk, D), lambda i: (i, 0)),
            pl.BlockSpec((B, D), lambda i: (0, 0)),
        ],
        out_specs=pl.BlockSpec(memory_space=pltpu.SMEM),
        out_shape=jax.ShapeDtypeStruct((1, 1), jnp.float32),
        interpret=interpret,
    )(c, e)
    return out[0, 0]


def kernel(inpt, trgs, emb_table, ffw_weight):
    inpt = inpt.astype(jnp.int32)
    trgs = trgs.astype(jnp.int32)
    e, c = _sc_gather(emb_table, inpt, ffw_weight, trgs)
    return _tc_loss(e, c)


# speedup vs baseline: 1.3680x; 1.3680x over previous
import functools

import jax
import jax.numpy as jnp
from jax import lax
from jax.experimental import pallas as pl
from jax.experimental.pallas import tpu as pltpu
from jax.experimental.pallas import tpu_sc as plsc


def _sc_gather(emb_table, inpt, ffw_weight, trgs):
    B = inpt.shape[0]
    D = emb_table.shape[1]
    info = plsc.get_sparse_core_info()
    nc, ns = info.num_cores, info.num_subcores
    nw = nc * ns
    b_per_w = B // nw
    mesh = plsc.VectorSubcoreMesh(core_axis_name="c", subcore_axis_name="s")

    @functools.partial(
        pl.kernel,
        mesh=mesh,
        out_type=[
            jax.ShapeDtypeStruct((B, D), jnp.float32),
            jax.ShapeDtypeStruct((B, D), jnp.float32),
        ],
        scratch_types=[
            pltpu.VMEM((b_per_w,), jnp.int32),
            pltpu.VMEM((b_per_w,), jnp.int32),
            pltpu.VMEM((b_per_w, D), jnp.float32),
            pltpu.VMEM((b_per_w, D), jnp.float32),
            pltpu.SemaphoreType.DMA,
        ],
    )
    def gather_kernel(emb_hbm, inpt_hbm, ffw_hbm, trgs_hbm, e_out, c_out,
                      idx_e, idx_c, rows_e, rows_c, sem):
        wid = lax.axis_index("s") * nc + lax.axis_index("c")
        base = wid * b_per_w
        pltpu.sync_copy(inpt_hbm.at[pl.ds(base, b_per_w)], idx_e)
        pltpu.sync_copy(trgs_hbm.at[pl.ds(base, b_per_w)], idx_c)

        def issue(g, _):
            ve = idx_e[pl.ds(g * 16, 16)]
            vc = idx_c[pl.ds(g * 16, 16)]
            for l in range(16):
                pltpu.async_copy(emb_hbm.at[ve[l]], rows_e.at[g * 16 + l], sem)
                pltpu.async_copy(ffw_hbm.at[vc[l]], rows_c.at[g * 16 + l], sem)
            return ()

        lax.fori_loop(0, b_per_w // 16, issue, ())
        pltpu.make_async_copy(emb_hbm.at[pl.ds(0, b_per_w)], rows_e, sem).wait()
        pltpu.make_async_copy(ffw_hbm.at[pl.ds(0, b_per_w)], rows_c, sem).wait()
        pltpu.sync_copy(rows_e, e_out.at[pl.ds(base, b_per_w)])
        pltpu.sync_copy(rows_c, c_out.at[pl.ds(base, b_per_w)])

    return gather_kernel(emb_table, inpt, ffw_weight, trgs)


def _tc_loss(e, c, interpret=False):
    B, D = e.shape
    blk = 512
    scale = 1.0 / (B * B)

    def body(c_ref, e_ref, out_ref):
        i = pl.program_id(0)
        lgt = lax.dot_general(
            c_ref[...], e_ref[...],
            (((1,), (1,)), ((), ())),
            preferred_element_type=jnp.float32,
        )
        part = jnp.sum(jnp.log(1.0 + jnp.exp(-lgt))) * scale

        @pl.when(i == 0)
        def _():
            out_ref[0, 0] = 0.0

        out_ref[0, 0] += part

    out = pl.pallas_call(
        body,
        grid=(B // blk,),
        in_specs=[
            pl.BlockSpec((blk, D), lambda i: (i, 0)),
            pl.BlockSpec((B, D), lambda i: (0, 0)),
        ],
        out_specs=pl.BlockSpec(memory_space=pltpu.SMEM),
        out_shape=jax.ShapeDtypeStruct((1, 1), jnp.float32),
        interpret=interpret,
    )(c, e)
    return out[0, 0]


def kernel(inpt, trgs, emb_table, ffw_weight):
    inpt = inpt.astype(jnp.int32)
    trgs = trgs.astype(jnp.int32)
    e, c = _sc_gather(emb_table, inpt, ffw_weight, trgs)
    return _tc_loss(e, c)
